# Initial kernel scaffold; baseline (speedup 1.0000x reference)
import jax
import jax.numpy as jnp
from jax.experimental import pallas as pl


def kernel(features, labels, centers):
    raise NotImplementedError("write your pallas kernel here")


# speedup vs baseline: 4.4698x; 4.4698x over previous
import functools

import jax
import jax.numpy as jnp
from jax import lax
from jax.experimental import pallas as pl
from jax.experimental.pallas import tpu as pltpu
from jax.experimental.pallas import tpu_sc as plsc

NUM_CLASSES = 10
FEAT_DIM = 2
ALPHA = 0.5
BATCH = 16384

NUM_TILES = 16
CHUNK = BATCH // NUM_TILES
LANES = 16
STEPS = CHUNK // LANES
PART = 3 * LANES


def _make_kernel():
    mesh = plsc.VectorSubcoreMesh(
        core_axis_name="c", subcore_axis_name="s", num_cores=1
    )

    @functools.partial(
        pl.kernel,
        mesh=mesh,
        compiler_params=pltpu.CompilerParams(needs_layout_passes=False),
        out_type=[
            jax.ShapeDtypeStruct((BATCH,), jnp.float32),
            jax.ShapeDtypeStruct((2 * LANES,), jnp.float32),
        ],
        scratch_types=[
            pltpu.VMEM((CHUNK,), jnp.int32),
            pltpu.VMEM((CHUNK * FEAT_DIM,), jnp.float32),
            pltpu.VMEM((CHUNK,), jnp.float32),
            pltpu.VMEM((LANES,), jnp.float32),
            pltpu.VMEM((LANES,), jnp.float32),
            pltpu.VMEM((PART,), jnp.float32),
            pltpu.VMEM((NUM_TILES * PART,), jnp.float32),
            pltpu.VMEM((2 * LANES,), jnp.float32),
            pltpu.VMEM_SHARED((NUM_TILES * PART,), jnp.float32),
        ],
    )
    def k(feat_hbm, lab_hbm, cen_hbm, res_hbm, nc_hbm,
          lab_v, feat_v, res_v, c0_v, c1_v, acc_v, all_v, nc_v, shared):
        wid = lax.axis_index("s")
        base = wid * CHUNK

        pltpu.sync_copy(lab_hbm.at[pl.ds(base, CHUNK)], lab_v)
        pltpu.sync_copy(
            feat_hbm.at[pl.ds(base * FEAT_DIM, CHUNK * FEAT_DIM)], feat_v
        )
        pltpu.sync_copy(cen_hbm.at[pl.ds(0, LANES)], c0_v)
        pltpu.sync_copy(cen_hbm.at[pl.ds(LANES, LANES)], c1_v)

        iota = lax.iota(jnp.int32, LANES)
        ones_f = jnp.ones((LANES,), jnp.float32)
        zeros_f = jnp.zeros((LANES,), jnp.float32)

        acc_v[pl.ds(0, LANES)] = zeros_f
        acc_v[pl.ds(LANES, LANES)] = zeros_f
        acc_v[pl.ds(2 * LANES, LANES)] = zeros_f

        for j in range(STEPS):
            off = j * LANES
            lab = lab_v[pl.ds(off, LANES)]
            ridx = iota + iota + off * FEAT_DIM
            f0 = plsc.load_gather(feat_v, [ridx])
            f1 = plsc.load_gather(feat_v, [ridx + 1])
            g0 = plsc.load_gather(c0_v, [lab])
            g1 = plsc.load_gather(c1_v, [lab])
            d0 = f0 - g0
            d1 = f1 - g1
            res_v[pl.ds(off, LANES)] = d0 * d0 + d1 * d1
            plsc.addupdate_scatter(acc_v, [lab], ones_f)
            plsc.addupdate_scatter(acc_v, [lab + LANES], f0)
            plsc.addupdate_scatter(acc_v, [lab + 2 * LANES], f1)

        pltpu.sync_copy(res_v, res_hbm.at[pl.ds(base, CHUNK)])

        pltpu.sync_copy(acc_v, shared.at[pl.ds(wid * PART, PART)])
        plsc.subcore_barrier()

        @pl.when(wid == 0)
        def _():
            pltpu.sync_copy(shared, all_v)
            cnt = zeros_f
            s0 = zeros_f
            s1 = zeros_f
            for t in range(NUM_TILES):
                cnt = cnt + all_v[pl.ds(t * PART, LANES)]
                s0 = s0 + all_v[pl.ds(t * PART + LANES, LANES)]
                s1 = s1 + all_v[pl.ds(t * PART + 2 * LANES, LANES)]
            c0 = c0_v[...]
            c1 = c1_v[...]
            scale = ALPHA / (cnt + 1.0)
            nc_v[pl.ds(0, LANES)] = c0 - (cnt * c0 - s0) * scale
            nc_v[pl.ds(LANES, LANES)] = c1 - (cnt * c1 - s1) * scale
            pltpu.sync_copy(nc_v, nc_hbm)

    return k


_sc_center_loss = _make_kernel()


@jax.jit
def kernel(features, labels, centers):
    labels = labels.reshape(-1).astype(jnp.int32)
    cen2 = jnp.zeros((2, LANES), jnp.float32).at[:, :NUM_CLASSES].set(centers.T)
    res, nc = _sc_center_loss(features.reshape(-1), labels, cen2.reshape(-1))
    return res.reshape(-1, 1), nc.reshape(2, LANES)[:, :NUM_CLASSES].T


# speedup vs baseline: 4.6001x; 1.0292x over previous
import functools

import jax
import jax.numpy as jnp
from jax import lax
from jax.experimental import pallas as pl
from jax.experimental.pallas import tpu as pltpu
from jax.experimental.pallas import tpu_sc as plsc

NUM_CLASSES = 10
FEAT_DIM = 2
ALPHA = 0.5
BATCH = 16384

NUM_TILES = 16
CHUNK = BATCH // NUM_TILES
LANES = 16
STEPS = CHUNK // LANES
PART = 3 * LANES


def _make_kernel():
    mesh = plsc.VectorSubcoreMesh(
        core_axis_name="c", subcore_axis_name="s", num_cores=1
    )

    @functools.partial(
        pl.kernel,
        mesh=mesh,
        compiler_params=pltpu.CompilerParams(needs_layout_passes=False),
        out_type=[
            jax.ShapeDtypeStruct((BATCH,), jnp.float32),
            jax.ShapeDtypeStruct((2 * LANES,), jnp.float32),
        ],
        scratch_types=[
            pltpu.VMEM((CHUNK,), jnp.int32),
            pltpu.VMEM((CHUNK * FEAT_DIM,), jnp.float32),
            pltpu.VMEM((CHUNK,), jnp.float32),
            pltpu.VMEM((LANES,), jnp.float32),
            pltpu.VMEM((LANES,), jnp.float32),
            pltpu.VMEM((PART,), jnp.float32),
            pltpu.VMEM((NUM_TILES * PART,), jnp.float32),
            pltpu.VMEM((2 * LANES,), jnp.float32),
            pltpu.VMEM_SHARED((NUM_TILES * PART,), jnp.float32),
        ],
    )
    def k(feat_hbm, lab_hbm, cen_hbm, res_hbm, nc_hbm,
          lab_v, feat_v, res_v, c0_v, c1_v, acc_v, all_v, nc_v, shared):
        wid = lax.axis_index("s")
        base = wid * CHUNK

        pltpu.sync_copy(lab_hbm.at[pl.ds(base, CHUNK)], lab_v)
        pltpu.sync_copy(
            feat_hbm.at[pl.ds(base * FEAT_DIM, CHUNK * FEAT_DIM)], feat_v
        )
        pltpu.sync_copy(cen_hbm.at[pl.ds(0, LANES)], c0_v)
        pltpu.sync_copy(cen_hbm.at[pl.ds(LANES, LANES)], c1_v)

        iota = lax.iota(jnp.int32, LANES)
        ones_f = jnp.ones((LANES,), jnp.float32)
        zeros_f = jnp.zeros((LANES,), jnp.float32)

        acc_v[pl.ds(0, LANES)] = zeros_f
        acc_v[pl.ds(LANES, LANES)] = zeros_f
        acc_v[pl.ds(2 * LANES, LANES)] = zeros_f

        def step(j, _):
            off = pl.multiple_of(j * LANES, LANES)
            lab = lab_v[pl.ds(off, LANES)]
            ridx = iota + iota + off * FEAT_DIM
            f0 = plsc.load_gather(feat_v, [ridx])
            f1 = plsc.load_gather(feat_v, [ridx + 1])
            g0 = plsc.load_gather(c0_v, [lab])
            g1 = plsc.load_gather(c1_v, [lab])
            d0 = f0 - g0
            d1 = f1 - g1
            res_v[pl.ds(off, LANES)] = d0 * d0 + d1 * d1
            plsc.addupdate_scatter(acc_v, [lab], ones_f)
            plsc.addupdate_scatter(acc_v, [lab + LANES], f0)
            plsc.addupdate_scatter(acc_v, [lab + 2 * LANES], f1)
            return 0

        lax.fori_loop(0, STEPS, step, 0)

        pltpu.sync_copy(res_v, res_hbm.at[pl.ds(base, CHUNK)])

        pltpu.sync_copy(acc_v, shared.at[pl.ds(wid * PART, PART)])
        plsc.subcore_barrier()

        @pl.when(wid == 0)
        def _():
            pltpu.sync_copy(shared, all_v)
            cnt = zeros_f
            s0 = zeros_f
            s1 = zeros_f
            for t in range(NUM_TILES):
                cnt = cnt + all_v[pl.ds(t * PART, LANES)]
                s0 = s0 + all_v[pl.ds(t * PART + LANES, LANES)]
                s1 = s1 + all_v[pl.ds(t * PART + 2 * LANES, LANES)]
            c0 = c0_v[...]
            c1 = c1_v[...]
            scale = ALPHA / (cnt + 1.0)
            nc_v[pl.ds(0, LANES)] = c0 - (cnt * c0 - s0) * scale
            nc_v[pl.ds(LANES, LANES)] = c1 - (cnt * c1 - s1) * scale
            pltpu.sync_copy(nc_v, nc_hbm)

    return k


_sc_center_loss = _make_kernel()


@jax.jit
def kernel(features, labels, centers):
    labels = labels.reshape(-1).astype(jnp.int32)
    cen2 = jnp.zeros((2, LANES), jnp.float32).at[:, :NUM_CLASSES].set(centers.T)
    res, nc = _sc_center_loss(features.reshape(-1), labels, cen2.reshape(-1))
    return res.reshape(-1, 1), nc.reshape(2, LANES)[:, :NUM_CLASSES].T


# speedup vs baseline: 4.7902x; 1.0413x over previous
import functools

import jax
import jax.numpy as jnp
from jax import lax
from jax.experimental import pallas as pl
from jax.experimental.pallas import tpu as pltpu
from jax.experimental.pallas import tpu_sc as plsc

NUM_CLASSES = 10
FEAT_DIM = 2
ALPHA = 0.5
BATCH = 16384

NUM_TILES = 16
CHUNK = BATCH // NUM_TILES
LANES = 16
STEPS = CHUNK // LANES
PART = 3 * LANES
CEN = NUM_CLASSES * FEAT_DIM


def _make_kernel():
    mesh = plsc.VectorSubcoreMesh(
        core_axis_name="c", subcore_axis_name="s", num_cores=1
    )

    @functools.partial(
        pl.kernel,
        mesh=mesh,
        compiler_params=pltpu.CompilerParams(needs_layout_passes=False),
        out_type=[
            jax.ShapeDtypeStruct((BATCH,), jnp.float32),
            jax.ShapeDtypeStruct((CEN,), jnp.float32),
        ],
        scratch_types=[
            pltpu.VMEM((CHUNK,), jnp.int32),
            pltpu.VMEM((CHUNK * FEAT_DIM,), jnp.float32),
            pltpu.VMEM((CHUNK,), jnp.float32),
            pltpu.VMEM((2 * LANES,), jnp.float32),
            pltpu.VMEM((PART,), jnp.float32),
            pltpu.VMEM((NUM_TILES * PART,), jnp.float32),
            pltpu.VMEM((2 * LANES,), jnp.float32),
            pltpu.VMEM_SHARED((NUM_TILES * PART,), jnp.float32),
        ],
    )
    def k(feat_hbm, lab_hbm, cen_hbm, res_hbm, nc_hbm,
          lab_v, feat_v, res_v, cen_v, acc_v, all_v, nc_v, shared):
        wid = lax.axis_index("s")
        base = wid * CHUNK

        pltpu.sync_copy(lab_hbm.at[pl.ds(base, CHUNK)], lab_v)
        pltpu.sync_copy(
            feat_hbm.at[pl.ds(base * FEAT_DIM, CHUNK * FEAT_DIM)], feat_v
        )
        pltpu.sync_copy(cen_hbm, cen_v.at[pl.ds(0, CEN)])

        iota = lax.iota(jnp.int32, LANES)
        ones_f = jnp.ones((LANES,), jnp.float32)
        zeros_f = jnp.zeros((LANES,), jnp.float32)

        acc_v[pl.ds(0, LANES)] = zeros_f
        acc_v[pl.ds(LANES, LANES)] = zeros_f
        acc_v[pl.ds(2 * LANES, LANES)] = zeros_f

        def step(j, _):
            off = pl.multiple_of(j * LANES, LANES)
            lab = lab_v[pl.ds(off, LANES)]
            lab2 = lab + lab
            ridx = iota + iota + off * FEAT_DIM
            f0 = plsc.load_gather(feat_v, [ridx])
            f1 = plsc.load_gather(feat_v, [ridx + 1])
            g0 = plsc.load_gather(cen_v, [lab2])
            g1 = plsc.load_gather(cen_v, [lab2 + 1])
            d0 = f0 - g0
            d1 = f1 - g1
            res_v[pl.ds(off, LANES)] = d0 * d0 + d1 * d1
            plsc.addupdate_scatter(acc_v, [lab], ones_f)
            plsc.addupdate_scatter(acc_v, [lab + LANES], f0)
            plsc.addupdate_scatter(acc_v, [lab + 2 * LANES], f1)
            return 0

        lax.fori_loop(0, STEPS, step, 0)

        pltpu.sync_copy(res_v, res_hbm.at[pl.ds(base, CHUNK)])

        pltpu.sync_copy(acc_v, shared.at[pl.ds(wid * PART, PART)])
        plsc.subcore_barrier()

        @pl.when(wid == 0)
        def _():
            pltpu.sync_copy(shared, all_v)
            cnt = zeros_f
            s0 = zeros_f
            s1 = zeros_f
            for t in range(NUM_TILES):
                cnt = cnt + all_v[pl.ds(t * PART, LANES)]
                s0 = s0 + all_v[pl.ds(t * PART + LANES, LANES)]
                s1 = s1 + all_v[pl.ds(t * PART + 2 * LANES, LANES)]
            valid = iota < NUM_CLASSES
            iota2 = iota + iota
            c0 = plsc.load_gather(cen_v, [iota2], mask=valid)
            c1 = plsc.load_gather(cen_v, [iota2 + 1], mask=valid)
            scale = ALPHA / (cnt + 1.0)
            n0 = c0 - (cnt * c0 - s0) * scale
            n1 = c1 - (cnt * c1 - s1) * scale
            plsc.store_scatter(nc_v, [iota2], n0, mask=valid)
            plsc.store_scatter(nc_v, [iota2 + 1], n1, mask=valid)
            pltpu.sync_copy(nc_v.at[pl.ds(0, CEN)], nc_hbm)

    return k


_sc_center_loss = _make_kernel()


@jax.jit
def kernel(features, labels, centers):
    res, nc = _sc_center_loss(
        features.reshape(-1), labels.reshape(-1), centers.reshape(-1)
    )
    return res.reshape(-1, 1), nc.reshape(NUM_CLASSES, FEAT_DIM)


# speedup vs baseline: 6.7807x; 1.4155x over previous
import functools

import jax
import jax.numpy as jnp
from jax import lax
from jax.experimental import pallas as pl
from jax.experimental.pallas import tpu as pltpu
from jax.experimental.pallas import tpu_sc as plsc

NUM_CLASSES = 10
FEAT_DIM = 2
ALPHA = 0.5
BATCH = 16384

NUM_TILES = 16
CHUNK = BATCH // NUM_TILES
LANES = 16
STEPS = CHUNK // LANES
PART = 3 * LANES
CEN = NUM_CLASSES * FEAT_DIM


def _make_kernel():
    mesh = plsc.VectorSubcoreMesh(
        core_axis_name="c", subcore_axis_name="s", num_cores=1
    )

    @functools.partial(
        pl.kernel,
        mesh=mesh,
        compiler_params=pltpu.CompilerParams(needs_layout_passes=False),
        out_type=[
            jax.ShapeDtypeStruct((BATCH,), jnp.float32),
            jax.ShapeDtypeStruct((CEN,), jnp.float32),
        ],
        scratch_types=[
            pltpu.VMEM((CHUNK,), jnp.int32),
            pltpu.VMEM((CHUNK * FEAT_DIM,), jnp.float32),
            pltpu.VMEM((CHUNK,), jnp.float32),
            pltpu.VMEM((2 * LANES,), jnp.float32),
            pltpu.VMEM((PART,), jnp.float32),
            pltpu.VMEM((NUM_TILES * PART,), jnp.float32),
            pltpu.VMEM((2 * LANES,), jnp.float32),
            pltpu.VMEM_SHARED((NUM_TILES * PART,), jnp.float32),
        ],
    )
    def k(feat_hbm, lab_hbm, cen_hbm, res_hbm, nc_hbm,
          lab_v, feat_v, res_v, cen_v, acc_v, all_v, nc_v, shared):
        wid = lax.axis_index("s")
        base = wid * CHUNK

        pltpu.sync_copy(lab_hbm.at[pl.ds(base, CHUNK)], lab_v)
        pltpu.sync_copy(feat_hbm.at[pl.ds(base, CHUNK)], feat_v.at[pl.ds(0, CHUNK)])
        pltpu.sync_copy(
            feat_hbm.at[pl.ds(BATCH + base, CHUNK)],
            feat_v.at[pl.ds(CHUNK, CHUNK)],
        )
        pltpu.sync_copy(cen_hbm, cen_v.at[pl.ds(0, CEN)])

        iota = lax.iota(jnp.int32, LANES)
        ones_f = jnp.ones((LANES,), jnp.float32)
        zeros_f = jnp.zeros((LANES,), jnp.float32)

        acc_v[pl.ds(0, LANES)] = zeros_f
        acc_v[pl.ds(LANES, LANES)] = zeros_f
        acc_v[pl.ds(2 * LANES, LANES)] = zeros_f

        def step(j, _):
            off = pl.multiple_of(j * LANES, LANES)
            lab = lab_v[pl.ds(off, LANES)]
            f0 = feat_v[pl.ds(off, LANES)]
            f1 = feat_v[pl.ds(CHUNK + off, LANES)]
            g0 = plsc.load_gather(cen_v, [lab])
            g1 = plsc.load_gather(cen_v, [lab + NUM_CLASSES])
            d0 = f0 - g0
            d1 = f1 - g1
            res_v[pl.ds(off, LANES)] = d0 * d0 + d1 * d1
            plsc.addupdate_scatter(acc_v, [lab], ones_f)
            plsc.addupdate_scatter(acc_v, [lab + LANES], f0)
            plsc.addupdate_scatter(acc_v, [lab + 2 * LANES], f1)
            return 0

        lax.fori_loop(0, STEPS, step, 0)

        pltpu.sync_copy(res_v, res_hbm.at[pl.ds(base, CHUNK)])

        pltpu.sync_copy(acc_v, shared.at[pl.ds(wid * PART, PART)])
        plsc.subcore_barrier()

        @pl.when(wid == 0)
        def _():
            pltpu.sync_copy(shared, all_v)
            cnt = zeros_f
            s0 = zeros_f
            s1 = zeros_f
            for t in range(NUM_TILES):
                cnt = cnt + all_v[pl.ds(t * PART, LANES)]
                s0 = s0 + all_v[pl.ds(t * PART + LANES, LANES)]
                s1 = s1 + all_v[pl.ds(t * PART + 2 * LANES, LANES)]
            valid = iota < NUM_CLASSES
            c0 = plsc.load_gather(cen_v, [iota], mask=valid)
            c1 = plsc.load_gather(cen_v, [iota + NUM_CLASSES], mask=valid)
            scale = ALPHA / (cnt + 1.0)
            n0 = c0 - (cnt * c0 - s0) * scale
            n1 = c1 - (cnt * c1 - s1) * scale
            plsc.store_scatter(nc_v, [iota], n0, mask=valid)
            plsc.store_scatter(nc_v, [iota + NUM_CLASSES], n1, mask=valid)
            pltpu.sync_copy(nc_v.at[pl.ds(0, CEN)], nc_hbm)

    return k


_sc_center_loss = _make_kernel()


@jax.jit
def kernel(features, labels, centers):
    res, nc = _sc_center_loss(
        features.T.reshape(-1), labels.reshape(-1), centers.T.reshape(-1)
    )
    return res.reshape(-1, 1), nc.reshape(FEAT_DIM, NUM_CLASSES).T


# speedup vs baseline: 6.7841x; 1.0005x over previous
import functools

import jax
import jax.numpy as jnp
from jax import lax
from jax.experimental import pallas as pl
from jax.experimental.pallas import tpu as pltpu
from jax.experimental.pallas import tpu_sc as plsc

NUM_CLASSES = 10
FEAT_DIM = 2
ALPHA = 0.5
BATCH = 16384

NUM_TILES = 16
CHUNK = BATCH // NUM_TILES
LANES = 16
STEPS = CHUNK // LANES
PART = 3 * LANES
CEN = NUM_CLASSES * FEAT_DIM


def _make_kernel():
    mesh = plsc.VectorSubcoreMesh(
        core_axis_name="c", subcore_axis_name="s", num_cores=1
    )

    @functools.partial(
        pl.kernel,
        mesh=mesh,
        compiler_params=pltpu.CompilerParams(needs_layout_passes=False),
        out_type=[
            jax.ShapeDtypeStruct((BATCH,), jnp.float32),
            jax.ShapeDtypeStruct((CEN,), jnp.float32),
        ],
        scratch_types=[
            pltpu.VMEM((CHUNK,), jnp.int32),
            pltpu.VMEM((CHUNK * FEAT_DIM,), jnp.float32),
            pltpu.VMEM((CHUNK,), jnp.float32),
            pltpu.VMEM((2 * LANES,), jnp.float32),
            pltpu.VMEM((2 * PART,), jnp.float32),
            pltpu.VMEM((NUM_TILES * PART,), jnp.float32),
            pltpu.VMEM((2 * LANES,), jnp.float32),
            pltpu.VMEM_SHARED((NUM_TILES * PART,), jnp.float32),
        ],
    )
    def k(feat_hbm, lab_hbm, cen_hbm, res_hbm, nc_hbm,
          lab_v, feat_v, res_v, cen_v, acc_v, all_v, nc_v, shared):
        wid = lax.axis_index("s")
        base = wid * CHUNK

        pltpu.sync_copy(lab_hbm.at[pl.ds(base, CHUNK)], lab_v)
        pltpu.sync_copy(feat_hbm.at[pl.ds(base, CHUNK)], feat_v.at[pl.ds(0, CHUNK)])
        pltpu.sync_copy(
            feat_hbm.at[pl.ds(BATCH + base, CHUNK)],
            feat_v.at[pl.ds(CHUNK, CHUNK)],
        )
        pltpu.sync_copy(cen_hbm, cen_v.at[pl.ds(0, CEN)])

        iota = lax.iota(jnp.int32, LANES)
        ones_f = jnp.ones((LANES,), jnp.float32)
        zeros_f = jnp.zeros((LANES,), jnp.float32)

        for r in range(6):
            acc_v[pl.ds(r * LANES, LANES)] = zeros_f

        def step(j, _):
            for u in range(2):
                off = pl.multiple_of(j * (2 * LANES) + u * LANES, LANES)
                bank = u * PART
                lab = lab_v[pl.ds(off, LANES)]
                f0 = feat_v[pl.ds(off, LANES)]
                f1 = feat_v[pl.ds(CHUNK + off, LANES)]
                g0 = plsc.load_gather(cen_v, [lab])
                g1 = plsc.load_gather(cen_v, [lab + NUM_CLASSES])
                d0 = f0 - g0
                d1 = f1 - g1
                res_v[pl.ds(off, LANES)] = d0 * d0 + d1 * d1
                plsc.addupdate_scatter(acc_v, [lab + bank], ones_f)
                plsc.addupdate_scatter(acc_v, [lab + (bank + LANES)], f0)
                plsc.addupdate_scatter(acc_v, [lab + (bank + 2 * LANES)], f1)
            return 0

        lax.fori_loop(0, STEPS // 2, step, 0)

        for r in range(3):
            acc_v[pl.ds(r * LANES, LANES)] = (
                acc_v[pl.ds(r * LANES, LANES)]
                + acc_v[pl.ds(PART + r * LANES, LANES)]
            )

        pltpu.sync_copy(res_v, res_hbm.at[pl.ds(base, CHUNK)])

        pltpu.sync_copy(acc_v.at[pl.ds(0, PART)], shared.at[pl.ds(wid * PART, PART)])
        plsc.subcore_barrier()

        @pl.when(wid == 0)
        def _():
            pltpu.sync_copy(shared, all_v)
            cnt = zeros_f
            s0 = zeros_f
            s1 = zeros_f
            for t in range(NUM_TILES):
                cnt = cnt + all_v[pl.ds(t * PART, LANES)]
                s0 = s0 + all_v[pl.ds(t * PART + LANES, LANES)]
                s1 = s1 + all_v[pl.ds(t * PART + 2 * LANES, LANES)]
            valid = iota < NUM_CLASSES
            c0 = plsc.load_gather(cen_v, [iota], mask=valid)
            c1 = plsc.load_gather(cen_v, [iota + NUM_CLASSES], mask=valid)
            scale = ALPHA / (cnt + 1.0)
            n0 = c0 - (cnt * c0 - s0) * scale
            n1 = c1 - (cnt * c1 - s1) * scale
            plsc.store_scatter(nc_v, [iota], n0, mask=valid)
            plsc.store_scatter(nc_v, [iota + NUM_CLASSES], n1, mask=valid)
            pltpu.sync_copy(nc_v.at[pl.ds(0, CEN)], nc_hbm)

    return k


_sc_center_loss = _make_kernel()


@jax.jit
def kernel(features, labels, centers):
    res, nc = _sc_center_loss(
        features.T.reshape(-1), labels.reshape(-1), centers.T.reshape(-1)
    )
    return res.reshape(-1, 1), nc.reshape(FEAT_DIM, NUM_CLASSES).T
